# Initial kernel scaffold; baseline (speedup 1.0000x reference)
import jax
import jax.numpy as jnp
from jax.experimental import pallas as pl


def kernel(node_features, node_attrs, edge_embedding, edge_attrs, edge_index, lin1_w0, lin1_w1, lin1_w2, mlp_w1, mlp_w2, head_a_lin2, head_b_lin2, head_a_sc, head_b_sc):
    raise NotImplementedError("write your pallas kernel here")


# speedup vs baseline: 4.8694x; 4.8694x over previous
import functools
import math

import jax
import jax.numpy as jnp
from jax import lax
from jax.experimental import pallas as pl
from jax.experimental.pallas import tpu as pltpu
from jax.experimental.pallas import tpu_sc as plsc

N_NODES_C = 10000
N_EDGES_C = 320000

_NC = 2
_NS = 16
_NW = _NC * _NS
_EB = 80
_EPT = N_EDGES_C // _NW
_NBLK = _EPT // _EB
_NFT = 10
_RPT = N_NODES_C // _NFT

_INV_SQRT3 = 1.0 / math.sqrt(3.0)
_INV_SQRT5 = 1.0 / math.sqrt(5.0)
_INV_SQRT32 = 1.0 / math.sqrt(32.0)


def _a1_body(nf_ref, w_ref, y_ref):
    y_ref[...] = lax.dot_general(
        nf_ref[...], w_ref[...], (((1,), (0,)), ((), ())),
        preferred_element_type=jnp.float32)


def _node_linear(node_features, w_big):
    blk = 1000
    grid = N_NODES_C // blk
    return pl.pallas_call(
        _a1_body,
        grid=(grid,),
        in_specs=[
            pl.BlockSpec((blk, 240), lambda i: (i, 0)),
            pl.BlockSpec((240, 256), lambda i: (0, 0)),
        ],
        out_specs=pl.BlockSpec((blk, 256), lambda i: (i, 0)),
        out_shape=jax.ShapeDtypeStruct((N_NODES_C, 256), jnp.float32),
    )(node_features, w_big)


def _a2_body(emb_ref, ea_ref, w1_ref, w2_ref, ew_ref):
    h = lax.dot_general(emb_ref[...], w1_ref[...], (((1,), (0,)), ((), ())),
                        preferred_element_type=jnp.float32)
    h = h * (1.0 / math.sqrt(8.0))
    h = h * (1.0 / (1.0 + jnp.exp(-h)))
    ew = lax.dot_general(h, w2_ref[...], (((1,), (0,)), ((), ())),
                         preferred_element_type=jnp.float32)
    ew = ew * (1.0 / math.sqrt(64.0))
    sh0 = ea_ref[:, 0:1]
    col = lax.broadcasted_iota(jnp.int32, ew.shape, 1)
    scale = jnp.where(col < 64, sh0, jnp.float32(1.0))
    scale = jnp.where(col < 112, scale, jnp.float32(0.0))
    ew_ref[...] = ew * scale


def _edge_mlp(edge_embedding, ea_pad, mlp_w1, mlp_w2):
    blk = 4000
    grid = N_EDGES_C // blk
    return pl.pallas_call(
        _a2_body,
        grid=(grid,),
        in_specs=[
            pl.BlockSpec((blk, 8), lambda i: (i, 0)),
            pl.BlockSpec((blk, 16), lambda i: (i, 0)),
            pl.BlockSpec((8, 64), lambda i: (0, 0)),
            pl.BlockSpec((64, 128), lambda i: (0, 0)),
        ],
        out_specs=pl.BlockSpec((blk, 128), lambda i: (i, 0)),
        out_shape=jax.ShapeDtypeStruct((N_EDGES_C, 128), jnp.float32),
    )(edge_embedding, ea_pad, mlp_w1, mlp_w2)


def _sc_edge_kernel(y_hbm, src_hbm, dst_hbm, ew_hbm, ea_hbm, zero_hbm, out_hbm,
                    src_v, dst_v, g_v, ew_v, ea_v, msg_v, agg, sem):
    c = lax.axis_index("c")
    s = lax.axis_index("s")
    wid = c * _NS + s

    @pl.when(s < _NFT)
    def _zero():
        pltpu.sync_copy(zero_hbm, agg.at[pl.ds(s * _RPT, _RPT)])
    plsc.subcore_barrier()

    base0 = wid * _EPT

    def pad_body(e, carry0):
        msg_v[e, pl.ds(112, 16)] = jnp.zeros((16,), jnp.float32)
        return carry0

    lax.fori_loop(0, _EB, pad_body, 0)

    def blk_body(b, carry):
        base = base0 + b * _EB
        pltpu.sync_copy(src_hbm.at[pl.ds(base, _EB)], src_v)
        pltpu.sync_copy(dst_hbm.at[pl.ds(base, _EB)], dst_v)
        pltpu.async_copy(y_hbm.at[src_v], g_v, sem).wait()
        pltpu.sync_copy(ew_hbm.at[pl.ds(base, _EB)], ew_v)
        pltpu.sync_copy(ea_hbm.at[pl.ds(base * 16, _EB * 16)], ea_v)

        def edge_body(e, carry2):
            shv = ea_v[pl.ds(e * 16, 16)]

            def splat(col):
                return shv[col]

            for c4 in range(4):
                sl = pl.ds(16 * c4, 16)
                msg_v[e, sl] = ew_v[e, sl] * g_v[e, sl]
            s10 = splat(1)
            s11 = splat(2)
            s12 = splat(3)
            for uc in range(2):
                acc = (s10 * g_v[e, pl.ds(64 + uc * 16, 16)]
                       + s11 * g_v[e, pl.ds(96 + uc * 16, 16)]
                       + s12 * g_v[e, pl.ds(128 + uc * 16, 16)])
                sl = pl.ds(64 + uc * 16, 16)
                msg_v[e, sl] = acc * ew_v[e, sl] * _INV_SQRT3
            acc2 = splat(4) * g_v[e, pl.ds(160, 16)]
            for i in range(1, 5):
                acc2 = acc2 + splat(4 + i) * g_v[e, pl.ds(160 + 16 * i, 16)]
            msg_v[e, pl.ds(96, 16)] = acc2 * ew_v[e, pl.ds(96, 16)] * _INV_SQRT5
            return carry2

        lax.fori_loop(0, _EB, edge_body, 0)
        pltpu.sync_copy(msg_v, agg.at[dst_v], add=True)
        return carry

    lax.fori_loop(0, _NBLK, blk_body, 0)
    plsc.subcore_barrier()

    @pl.when(s < _NFT)
    def _flush():
        pltpu.sync_copy(agg.at[pl.ds(s * _RPT, _RPT)],
                        out_hbm.at[c, pl.ds(s * _RPT, _RPT)])


def _sc_edge_stage(y, src, dst, ew, ea_pad, zero_init):
    mesh = plsc.VectorSubcoreMesh(core_axis_name="c", subcore_axis_name="s")
    fn = functools.partial(
        pl.kernel,
        mesh=mesh,
        out_type=jax.ShapeDtypeStruct((_NC, N_NODES_C, 128), jnp.float32),
        scratch_types=[
            pltpu.VMEM((_EB,), jnp.int32),
            pltpu.VMEM((_EB,), jnp.int32),
            pltpu.VMEM((_EB, 256), jnp.float32),
            pltpu.VMEM((_EB, 128), jnp.float32),
            pltpu.VMEM((_EB * 16,), jnp.float32),
            pltpu.VMEM((_EB, 128), jnp.float32),
            pltpu.VMEM_SHARED((N_NODES_C, 128), jnp.float32),
            pltpu.SemaphoreType.DMA,
        ],
    )(_sc_edge_kernel)
    return fn(y, src, dst, ew, ea_pad, zero_init)


def _c_body(p_ref, x0_ref, na_ref, ha_ref, hb_ref, wa_ref, wb_ref, out_ref):
    agg = (p_ref[0, :, :112] + p_ref[1, :, :112]) * _INV_SQRT32
    za = lax.dot_general(agg[:, :96], ha_ref[...], (((1,), (0,)), ((), ())),
                         preferred_element_type=jnp.float32)
    za = za * (1.0 / math.sqrt(96.0))
    zb = lax.dot_general(agg, hb_ref[...], (((1,), (0,)), ((), ())),
                         preferred_element_type=jnp.float32)
    zb = zb * (1.0 / math.sqrt(112.0))
    x0 = x0_ref[...]
    ta = lax.dot_general(x0, wa_ref[...], (((1,), (0,)), ((), ())),
                         preferred_element_type=jnp.float32)
    tb = lax.dot_general(x0, wb_ref[...], (((1,), (0,)), ((), ())),
                         preferred_element_type=jnp.float32)
    na = na_ref[...]
    sc_a = na[:, 0:1] * ta[:, 0:64]
    sc_b = na[:, 0:1] * tb[:, 0:64]
    for v in range(1, 4):
        sc_a = sc_a + na[:, v:v + 1] * ta[:, 64 * v:64 * (v + 1)]
        sc_b = sc_b + na[:, v:v + 1] * tb[:, 64 * v:64 * (v + 1)]
    inv16 = 1.0 / 16.0
    pa = za + sc_a * inv16
    pb = zb + sc_b * inv16
    out_a = pa * (1.0 / (1.0 + jnp.exp(-pa)))
    out_b = pb * (1.0 / (1.0 + jnp.exp(-pb)))
    out_ref[:, :64] = out_a
    out_ref[:, 64:] = out_b


def _heads(partials, x0, node_attrs, ha, hb, wa, wb):
    blk = 1000
    grid = N_NODES_C // blk
    return pl.pallas_call(
        _c_body,
        grid=(grid,),
        in_specs=[
            pl.BlockSpec((2, blk, 128), lambda i: (0, i, 0)),
            pl.BlockSpec((blk, 64), lambda i: (i, 0)),
            pl.BlockSpec((blk, 4), lambda i: (i, 0)),
            pl.BlockSpec((96, 64), lambda i: (0, 0)),
            pl.BlockSpec((112, 64), lambda i: (0, 0)),
            pl.BlockSpec((64, 256), lambda i: (0, 0)),
            pl.BlockSpec((64, 256), lambda i: (0, 0)),
        ],
        out_specs=pl.BlockSpec((blk, 128), lambda i: (i, 0)),
        out_shape=jax.ShapeDtypeStruct((N_NODES_C, 128), jnp.float32),
    )(partials, x0, node_attrs, ha, hb, wa, wb)


def _build_w_big(lin1_w0, lin1_w1, lin1_w2):
    eye3 = jnp.eye(3, dtype=jnp.float32)
    eye5 = jnp.eye(5, dtype=jnp.float32)
    b1 = jnp.einsum('uv,ij->uijv', lin1_w1, eye3).reshape(96, 96)
    b2 = jnp.einsum('uv,ij->uijv', lin1_w2, eye5).reshape(80, 80)
    w = jnp.zeros((240, 256), jnp.float32)
    w = w.at[0:64, 0:64].set(lin1_w0 / 8.0)
    w = w.at[64:160, 64:160].set(b1 * _INV_SQRT32)
    w = w.at[160:240, 160:240].set(b2 / 4.0)
    return w


def kernel(node_features, node_attrs, edge_embedding, edge_attrs, edge_index,
           lin1_w0, lin1_w1, lin1_w2, mlp_w1, mlp_w2,
           head_a_lin2, head_b_lin2, head_a_sc, head_b_sc):
    w_big = _build_w_big(lin1_w0, lin1_w1, lin1_w2)
    ea_pad = jnp.pad(edge_attrs, ((0, 0), (0, 7)))
    src = edge_index[0].astype(jnp.int32)
    dst = edge_index[1].astype(jnp.int32)
    zero_init = jnp.zeros((_RPT, 128), jnp.float32)
    mlp_w2p = jnp.pad(mlp_w2, ((0, 0), (0, 16)))
    wa = head_a_sc.reshape(64, 256)
    wb = head_b_sc.reshape(64, 256)

    y = _node_linear(node_features, w_big)
    ew = _edge_mlp(edge_embedding, ea_pad, mlp_w1, mlp_w2p)
    partials = _sc_edge_stage(y, src, dst, ew, ea_pad.reshape(-1), zero_init)
    out = _heads(partials, node_features[:, :64], node_attrs,
                 head_a_lin2, head_b_lin2, wa, wb)
    return out
